# Initial kernel scaffold; baseline (speedup 1.0000x reference)
import jax
import jax.numpy as jnp
from jax.experimental import pallas as pl


def kernel(signal, cell_idx, num_cells):
    raise NotImplementedError("write your pallas kernel here")


# speedup vs baseline: 2.2438x; 2.2438x over previous
import jax
import jax.numpy as jnp
from jax import lax
from jax.experimental import pallas as pl
from jax.experimental.pallas import tpu as pltpu
from jax.experimental.pallas import tpu_sc as plsc

N = 320000
D = 128
S = 10000
NC = 2
NS = 16
C = 80
NCHUNK = N // C
CPT = NCHUNK // NS
HALF = 5120
SPC = 5248
SPT = SPC // NS
ZR = 8
CNTW = 16


def _pool_body(sig_hbm, idx_hbm, psum_hbm, pcnt_hbm,
               idx_b, idx_r, rows_v, ones_v, zsum_v, zcnt_v,
               ssum, scnt):
    c = lax.axis_index("c")
    s = lax.axis_index("s")

    for r in range(ZR):
        for jj in range(D // 16):
            zsum_v[r, pl.ds(jj * 16, 16)] = jnp.zeros((16,), jnp.float32)
        zcnt_v[r, :] = jnp.zeros((16,), jnp.float32)

    def fill_ones(r, carry):
        ones_v[r, :] = jnp.ones((16,), jnp.float32)
        return carry
    lax.fori_loop(0, C, fill_ones, 0)

    segbase = s * SPT

    def zstep(k, carry):
        pltpu.sync_copy(zsum_v, ssum.at[pl.ds(segbase + k * ZR, ZR)])
        pltpu.sync_copy(zcnt_v, scnt.at[pl.ds(segbase + k * ZR, ZR)])
        return carry

    lax.fori_loop(0, SPT // ZR, zstep, 0)
    plsc.subcore_barrier()

    lo = c * HALF

    def step(k, carry):
        chunk = s + k * NS
        pltpu.sync_copy(idx_hbm.at[chunk], idx_b)
        pltpu.sync_copy(sig_hbm.at[pl.ds(chunk * C, C)], rows_v)
        for jj in range(C // 16):
            v = idx_b[0, pl.ds(jj * 16, 16)]
            local = v - lo
            ok = (local >= 0) & (local < HALF)
            idx_r[pl.ds(jj * 16, 16)] = jnp.where(ok, local, HALF)
        pltpu.sync_copy(rows_v, ssum.at[idx_r], add=True)
        pltpu.sync_copy(ones_v, scnt.at[idx_r], add=True)
        return carry

    lax.fori_loop(0, CPT, step, 0)

    plsc.subcore_barrier()
    pltpu.sync_copy(ssum.at[pl.ds(segbase, SPT)],
                    psum_hbm.at[c, pl.ds(segbase, SPT)])
    pltpu.sync_copy(scnt.at[pl.ds(segbase, SPT)],
                    pcnt_hbm.at[c, pl.ds(segbase, SPT)])


_pool = pl.kernel(
    _pool_body,
    out_type=(jax.ShapeDtypeStruct((NC, SPC, D), jnp.float32),
              jax.ShapeDtypeStruct((NC, SPC, CNTW), jnp.float32)),
    mesh=plsc.VectorSubcoreMesh(core_axis_name="c", subcore_axis_name="s"),
    scratch_types=[
        pltpu.VMEM((1, C), jnp.int32),
        pltpu.VMEM((C,), jnp.int32),
        pltpu.VMEM((C, D), jnp.float32),
        pltpu.VMEM((C, CNTW), jnp.float32),
        pltpu.VMEM((ZR, D), jnp.float32),
        pltpu.VMEM((ZR, CNTW), jnp.float32),
        pltpu.VMEM_SHARED((SPC, D), jnp.float32),
        pltpu.VMEM_SHARED((SPC, CNTW), jnp.float32),
    ],
)


RB = 1280


def _combine_body(p_ref, c_ref, o_ref):
    sums = p_ref[0]
    cnt = c_ref[0]
    o_ref[...] = sums / jnp.maximum(cnt[:, 0:1], 1.0)


_combine = pl.pallas_call(
    _combine_body,
    grid=(pl.cdiv(S, RB),),
    in_specs=[
        pl.BlockSpec((1, RB, D), lambda i: (i // (HALF // RB), i % (HALF // RB), 0)),
        pl.BlockSpec((1, RB, CNTW), lambda i: (i // (HALF // RB), i % (HALF // RB), 0)),
    ],
    out_specs=pl.BlockSpec((RB, D), lambda i: (i, 0)),
    out_shape=jax.ShapeDtypeStruct((S, D), jnp.float32),
)


@jax.jit
def _impl(signal, cell_idx):
    idx3 = cell_idx.reshape(NCHUNK, 1, C)
    psum, pcnt = _pool(signal, idx3)
    return _combine(psum, pcnt)


def kernel(signal, cell_idx, num_cells):
    return _impl(signal, cell_idx)


# speedup vs baseline: 3.9069x; 1.7412x over previous
import jax
import jax.numpy as jnp
from jax import lax
from jax.experimental import pallas as pl
from jax.experimental.pallas import tpu as pltpu
from jax.experimental.pallas import tpu_sc as plsc

N = 320000
D = 128
S = 10000
NC = 2
NS = 16
C = 80
NCHUNK = N // C
CPT = NCHUNK // NS
HALF = 5120
SPC = 5248
SPT = SPC // NS
ZR = 8
CNTW = 128


def _bounds_body(i_ref, o_ref):
    b = jnp.sum((i_ref[...] < HALF).astype(jnp.int32))
    bceil = (b + C - 1) // C
    bfloor = b // C
    sub = lax.broadcasted_iota(jnp.int32, (8, 128), 0)
    lane = lax.broadcasted_iota(jnp.int32, (8, 128), 1)
    o_ref[...] = jnp.where((sub == 0) & (lane == 0), bceil,
                           jnp.where((sub == 0) & (lane == 1), bfloor, 0))


_bounds = pl.pallas_call(
    _bounds_body,
    out_shape=jax.ShapeDtypeStruct((8, 128), jnp.int32),
)


def _pool_body(sig_hbm, idx_hbm, bnd_hbm, psum_hbm, pcnt_hbm,
               idx_b, idx_r, rows_v, ones_v, zsum_v, zcnt_v, bnd_v,
               ssum, scnt):
    c = lax.axis_index("c")
    s = lax.axis_index("s")

    for r in range(ZR):
        for jj in range(D // 16):
            zsum_v[r, pl.ds(jj * 16, 16)] = jnp.zeros((16,), jnp.float32)
        for jj in range(CNTW // 16):
            zcnt_v[r, pl.ds(jj * 16, 16)] = jnp.zeros((16,), jnp.float32)

    def fill_ones(r, carry):
        for jj in range(CNTW // 16):
            ones_v[r, pl.ds(jj * 16, 16)] = jnp.ones((16,), jnp.float32)
        return carry
    lax.fori_loop(0, C, fill_ones, 0)

    segbase = s * SPT

    def zstep(k, carry):
        pltpu.sync_copy(zsum_v, ssum.at[pl.ds(segbase + k * ZR, ZR)])
        pltpu.sync_copy(zcnt_v, scnt.at[pl.ds(segbase + k * ZR, ZR)])
        return carry

    lax.fori_loop(0, SPT // ZR, zstep, 0)
    plsc.subcore_barrier()

    pltpu.sync_copy(bnd_hbm, bnd_v)
    bv = bnd_v[0, pl.ds(0, 16)]
    bceil = bv[0]
    bfloor = bv[1]
    ci = jnp.int32(c)
    cb_lo = bfloor * ci
    cb_hi = bceil + (NCHUNK - bceil) * ci
    k_lo = lax.div(jnp.maximum(cb_lo - s + NS - 1, 0), NS)
    k_hi = lax.div(jnp.maximum(cb_hi - s + NS - 1, 0), NS)

    lo = c * HALF

    def step(k, carry):
        chunk = s + k * NS
        pltpu.sync_copy(idx_hbm.at[chunk], idx_b)
        pltpu.sync_copy(sig_hbm.at[pl.ds(chunk * C, C)], rows_v)
        for jj in range(C // 16):
            v = idx_b[0, pl.ds(jj * 16, 16)]
            local = v - lo
            ok = (local >= 0) & (local < HALF)
            idx_r[pl.ds(jj * 16, 16)] = jnp.where(ok, local, HALF)
        pltpu.sync_copy(rows_v, ssum.at[idx_r], add=True)
        pltpu.sync_copy(ones_v, scnt.at[idx_r], add=True)
        return carry

    lax.fori_loop(k_lo, k_hi, step, 0)

    plsc.subcore_barrier()
    pltpu.sync_copy(ssum.at[pl.ds(segbase, SPT)],
                    psum_hbm.at[c, pl.ds(segbase, SPT)])
    pltpu.sync_copy(scnt.at[pl.ds(segbase, SPT)],
                    pcnt_hbm.at[c, pl.ds(segbase, SPT)])


_pool = pl.kernel(
    _pool_body,
    out_type=(jax.ShapeDtypeStruct((NC, SPC, D), jnp.float32),
              jax.ShapeDtypeStruct((NC, SPC, CNTW), jnp.float32)),
    mesh=plsc.VectorSubcoreMesh(core_axis_name="c", subcore_axis_name="s"),
    scratch_types=[
        pltpu.VMEM((1, C), jnp.int32),
        pltpu.VMEM((C,), jnp.int32),
        pltpu.VMEM((C, D), jnp.float32),
        pltpu.VMEM((C, CNTW), jnp.float32),
        pltpu.VMEM((ZR, D), jnp.float32),
        pltpu.VMEM((ZR, CNTW), jnp.float32),
        pltpu.VMEM((8, 128), jnp.int32),
        pltpu.VMEM_SHARED((SPC, D), jnp.float32),
        pltpu.VMEM_SHARED((SPC, CNTW), jnp.float32),
    ],
)


RB = 1280


def _combine_body(p_ref, c_ref, o_ref):
    sums = p_ref[0]
    cnt = c_ref[0]
    o_ref[...] = sums / jnp.maximum(cnt[:, 0:1], 1.0)


_combine = pl.pallas_call(
    _combine_body,
    grid=(pl.cdiv(S, RB),),
    in_specs=[
        pl.BlockSpec((1, RB, D), lambda i: (i // (HALF // RB), i % (HALF // RB), 0)),
        pl.BlockSpec((1, RB, CNTW), lambda i: (i // (HALF // RB), i % (HALF // RB), 0)),
    ],
    out_specs=pl.BlockSpec((RB, D), lambda i: (i, 0)),
    out_shape=jax.ShapeDtypeStruct((S, D), jnp.float32),
)


@jax.jit
def _impl(signal, cell_idx):
    bnd = _bounds(cell_idx.reshape(NCHUNK, C))
    idx3 = cell_idx.reshape(NCHUNK, 1, C)
    psum, pcnt = _pool(signal, idx3, bnd)
    return _combine(psum, pcnt)


def kernel(signal, cell_idx, num_cells):
    return _impl(signal, cell_idx)


# speedup vs baseline: 6.7790x; 1.7352x over previous
import jax
import jax.numpy as jnp
from jax import lax
from jax.experimental import pallas as pl
from jax.experimental.pallas import tpu as pltpu
from jax.experimental.pallas import tpu_sc as plsc

N = 320000
D = 128
S = 10000
NC = 2
NS = 16
C = 80
NCHUNK = N // C
CPT = NCHUNK // NS
HALF = 5120
SPC = 5248
SPT = SPC // NS
ZR = 8
CNTW = 128


def _bounds_body(i_ref, o_ref):
    b = jnp.sum((i_ref[...] < HALF).astype(jnp.int32))
    bceil = (b + C - 1) // C
    bfloor = b // C
    sub = lax.broadcasted_iota(jnp.int32, (8, 128), 0)
    lane = lax.broadcasted_iota(jnp.int32, (8, 128), 1)
    o_ref[...] = jnp.where((sub == 0) & (lane == 0), bceil,
                           jnp.where((sub == 0) & (lane == 1), bfloor, 0))


_bounds = pl.pallas_call(
    _bounds_body,
    out_shape=jax.ShapeDtypeStruct((8, 128), jnp.int32),
)


def _pool_body(sig_hbm, idx_hbm, bnd_hbm, psum_hbm, pcnt_hbm,
               idx_b, idx_b1, idx_r, idx_r1, rows_v, rows1, ones_v, zsum_v,
               zcnt_v, bnd_v, ssum, scnt):
    c = lax.axis_index("c")
    s = lax.axis_index("s")

    for r in range(ZR):
        for jj in range(D // 16):
            zsum_v[r, pl.ds(jj * 16, 16)] = jnp.zeros((16,), jnp.float32)
        for jj in range(CNTW // 16):
            zcnt_v[r, pl.ds(jj * 16, 16)] = jnp.zeros((16,), jnp.float32)

    def fill_ones(r, carry):
        for jj in range(CNTW // 16):
            ones_v[r, pl.ds(jj * 16, 16)] = jnp.ones((16,), jnp.float32)
        return carry
    lax.fori_loop(0, C, fill_ones, 0)

    segbase = s * SPT

    def zstep(k, carry):
        pltpu.sync_copy(zsum_v, ssum.at[pl.ds(segbase + k * ZR, ZR)])
        pltpu.sync_copy(zcnt_v, scnt.at[pl.ds(segbase + k * ZR, ZR)])
        return carry

    lax.fori_loop(0, SPT // ZR, zstep, 0)
    plsc.subcore_barrier()

    pltpu.sync_copy(bnd_hbm, bnd_v)
    bv = bnd_v[0, pl.ds(0, 16)]
    bceil = bv[0]
    bfloor = bv[1]
    ci = jnp.int32(c)
    cb_lo = bfloor * ci
    cb_hi = bceil + (NCHUNK - bceil) * ci
    k_lo = lax.div(jnp.maximum(cb_lo - s + NS - 1, 0), NS)
    k_hi = lax.div(jnp.maximum(cb_hi - s + NS - 1, 0), NS)

    lo = c * HALF

    T = k_hi - k_lo
    odd = T - 2 * lax.div(T, 2)
    k_lo = k_lo - odd * ci
    k_hi = k_hi + odd * (1 - ci)
    npair = lax.div(k_hi - k_lo, 2)
    maxslot = NCHUNK // NS - 1

    def remap_scatter(ib, ir, rb):
        for jj in range(C // 16):
            v = ib[0, pl.ds(jj * 16, 16)]
            local = v - lo
            ok = (local >= 0) & (local < HALF)
            ir[pl.ds(jj * 16, 16)] = jnp.where(ok, local, HALF)
        pltpu.sync_copy(rb, ssum.at[ir], add=True)
        pltpu.sync_copy(ones_v, scnt.at[ir], add=True)

    def main(si0, si1, sr0, sr1):
        def load(slot, ib, rb, si, sr):
            chunk = s + jnp.minimum(slot, maxslot) * NS
            pltpu.async_copy(idx_hbm.at[chunk], ib, si)
            pltpu.async_copy(sig_hbm.at[pl.ds(chunk * C, C)], rb, sr)

        def wait(slot, ib, rb, si, sr):
            chunk = s + jnp.minimum(slot, maxslot) * NS
            pltpu.make_async_copy(idx_hbm.at[chunk], ib, si).wait()
            pltpu.make_async_copy(
                sig_hbm.at[pl.ds(chunk * C, C)], rb, sr).wait()

        load(k_lo, idx_b, rows_v, si0, sr0)

        def step(t, carry):
            k = k_lo + 2 * t
            load(k + 1, idx_b1, rows1, si1, sr1)
            wait(k, idx_b, rows_v, si0, sr0)
            remap_scatter(idx_b, idx_r, rows_v)
            load(k + 2, idx_b, rows_v, si0, sr0)
            wait(k + 1, idx_b1, rows1, si1, sr1)
            remap_scatter(idx_b1, idx_r1, rows1)
            return carry

        lax.fori_loop(0, npair, step, 0)
        wait(k_lo + 2 * npair, idx_b, rows_v, si0, sr0)

    pl.run_scoped(main,
                  pltpu.SemaphoreType.DMA, pltpu.SemaphoreType.DMA,
                  pltpu.SemaphoreType.DMA, pltpu.SemaphoreType.DMA)

    plsc.subcore_barrier()
    pltpu.sync_copy(ssum.at[pl.ds(segbase, SPT)],
                    psum_hbm.at[c, pl.ds(segbase, SPT)])
    pltpu.sync_copy(scnt.at[pl.ds(segbase, SPT)],
                    pcnt_hbm.at[c, pl.ds(segbase, SPT)])


_pool = pl.kernel(
    _pool_body,
    out_type=(jax.ShapeDtypeStruct((NC, SPC, D), jnp.float32),
              jax.ShapeDtypeStruct((NC, SPC, CNTW), jnp.float32)),
    mesh=plsc.VectorSubcoreMesh(core_axis_name="c", subcore_axis_name="s"),
    scratch_types=[
        pltpu.VMEM((1, C), jnp.int32),
        pltpu.VMEM((1, C), jnp.int32),
        pltpu.VMEM((C,), jnp.int32),
        pltpu.VMEM((C,), jnp.int32),
        pltpu.VMEM((C, D), jnp.float32),
        pltpu.VMEM((C, D), jnp.float32),
        pltpu.VMEM((C, CNTW), jnp.float32),
        pltpu.VMEM((ZR, D), jnp.float32),
        pltpu.VMEM((ZR, CNTW), jnp.float32),
        pltpu.VMEM((8, 128), jnp.int32),
        pltpu.VMEM_SHARED((SPC, D), jnp.float32),
        pltpu.VMEM_SHARED((SPC, CNTW), jnp.float32),
    ],
)


RB = 1280


def _combine_body(p_ref, c_ref, o_ref):
    sums = p_ref[0]
    cnt = c_ref[0]
    o_ref[...] = sums / jnp.maximum(cnt[:, 0:1], 1.0)


_combine = pl.pallas_call(
    _combine_body,
    grid=(pl.cdiv(S, RB),),
    in_specs=[
        pl.BlockSpec((1, RB, D), lambda i: (i // (HALF // RB), i % (HALF // RB), 0)),
        pl.BlockSpec((1, RB, CNTW), lambda i: (i // (HALF // RB), i % (HALF // RB), 0)),
    ],
    out_specs=pl.BlockSpec((RB, D), lambda i: (i, 0)),
    out_shape=jax.ShapeDtypeStruct((S, D), jnp.float32),
)


@jax.jit
def _impl(signal, cell_idx):
    bnd = _bounds(cell_idx.reshape(NCHUNK, C))
    idx3 = cell_idx.reshape(NCHUNK, 1, C)
    psum, pcnt = _pool(signal, idx3, bnd)
    return _combine(psum, pcnt)


def kernel(signal, cell_idx, num_cells):
    return _impl(signal, cell_idx)
